# baseline (speedup 1.0000x reference)
import functools

import jax
import jax.numpy as jnp
from jax import lax
from jax.experimental import pallas as pl
from jax.experimental.pallas import tpu as pltpu
from jax.experimental.pallas import tpu_sc as plsc

RESO = 512
CHAN = 32
M = 1000000

NC = 2
NS = 16
L = 16
NW = NC * NS

CH = 80
NCH = M // CH
ITERS = -(-NCH // NW)
NSTEP = ITERS + 1
NSRC = 12

_mesh = plsc.VectorSubcoreMesh(core_axis_name="c", subcore_axis_name="s")


@functools.partial(
    pl.kernel,
    mesh=_mesh,
    out_type=jax.ShapeDtypeStruct((M * CHAN,), jnp.float32),
    scratch_types=[
        pltpu.VMEM((3 * CH,), jnp.float32),
        pltpu.VMEM((2, NSRC, CH), jnp.int32),
        pltpu.VMEM((2, CH * 17), jnp.float32),
        pltpu.VMEM((2, NSRC, CH, CHAN), jnp.float32),
        pltpu.VMEM((CH * CHAN,), jnp.float32),
        pltpu.SemaphoreType.DMA,
        pltpu.SemaphoreType.DMA,
    ],
    compiler_params=pltpu.CompilerParams(needs_layout_passes=False,
                                         use_tc_tiling_on_sc=False),
)
def _tri_sc(planes_hbm, xyzp_hbm, out_hbm, xyz_v, idx_v, w_v,
            rows_v, out_v, sem0, sem1):
    wid = lax.axis_index("s") * NC + lax.axis_index("c")
    sems = (sem0, sem1)
    stride16 = lax.iota(jnp.int32, L) * CHAN
    iota16 = lax.iota(jnp.int32, L)

    def bf16_round(v):
        u = lax.bitcast_convert_type(v, jnp.uint32)
        u = ((u + jnp.uint32(0x7FFF) + ((u >> 16) & jnp.uint32(1)))
             & jnp.uint32(0xFFFF0000))
        return lax.bitcast_convert_type(u, jnp.float32)

    def fire(it, b):
        ch = it * NW + wid

        @pl.when(ch < NCH)
        def _():
            base = ch * CH
            pltpu.sync_copy(xyzp_hbm.at[ch], xyz_v)

            for g in range(CH // L):
                sl = pl.ds(g * L, L)
                x = bf16_round(xyz_v[pl.ds(0 * CH + g * L, L)])
                y = bf16_round(xyz_v[pl.ds(1 * CH + g * L, L)])
                z = bf16_round(xyz_v[pl.ds(2 * CH + g * L, L)])
                for p, (gx, gy) in enumerate(((y, x), (z, x), (y, z))):
                    colf = (gx + 1.0) * (0.5 * (RESO - 1))
                    rowf = (gy + 1.0) * (0.5 * (RESO - 1))
                    c0 = jnp.clip(colf.astype(jnp.int32), 0, RESO - 2)
                    r0 = jnp.clip(rowf.astype(jnp.int32), 0, RESO - 2)
                    fc = colf - c0.astype(jnp.float32)
                    fr = rowf - r0.astype(jnp.float32)
                    base_i = r0 * RESO + c0 + (p * RESO * RESO)
                    idx_v[b, 4 * p + 0, sl] = base_i
                    idx_v[b, 4 * p + 1, sl] = base_i + 1
                    idx_v[b, 4 * p + 2, sl] = base_i + RESO
                    idx_v[b, 4 * p + 3, sl] = base_i + RESO + 1
                    wsc = iota16 * 17 + (g * L * 17 + 4 * p)
                    plsc.store_scatter(w_v.at[b], [wsc],
                                       (1.0 - fc) * (1.0 - fr))
                    plsc.store_scatter(w_v.at[b], [wsc + 1], fc * (1.0 - fr))
                    plsc.store_scatter(w_v.at[b], [wsc + 2], (1.0 - fc) * fr)
                    plsc.store_scatter(w_v.at[b], [wsc + 3], fc * fr)

            for s in range(NSRC):
                pltpu.async_copy(planes_hbm.at[idx_v.at[b, s]],
                                 rows_v.at[b, s], sems[b])

    def drain_acc(it, b):
        ch = it * NW + wid

        @pl.when(jnp.logical_and(it >= 0, ch < NCH))
        def _():
            base = ch * CH
            for s in range(NSRC):
                pltpu.make_async_copy(planes_hbm.at[idx_v.at[b, s]],
                                      rows_v.at[b, s], sems[b]).wait()

            def pt_body(pt, carry2):
                o = pt * CHAN
                wrow = w_v[b, pl.ds(pt * 17, L)]
                w = wrow[0]
                acc0 = w * rows_v[b, 0, pt, pl.ds(0, L)]
                acc1 = w * rows_v[b, 0, pt, pl.ds(L, L)]
                for s in range(1, NSRC):
                    w = wrow[s]
                    acc0 = acc0 + w * rows_v[b, s, pt, pl.ds(0, L)]
                    acc1 = acc1 + w * rows_v[b, s, pt, pl.ds(L, L)]
                out_v[pl.ds(o, L)] = acc0
                out_v[pl.ds(o + L, L)] = acc1
                return carry2

            lax.fori_loop(0, CH, pt_body, 0, unroll=2)

            pltpu.sync_copy(out_v, out_hbm.at[pl.ds(base * CHAN, CH * CHAN)])

    def pair_body(it2, carry):
        for parity in range(2):
            step = it2 * 2 + parity
            fire(step, parity)
            drain_acc(step - 1, 1 - parity)
        return carry

    lax.fori_loop(0, NSTEP // 2, pair_body, 0)


def kernel(xyz, triplane):
    planes = jnp.transpose(triplane, (0, 2, 3, 1)).reshape(3 * RESO * RESO,
                                                           CHAN)
    xyzp = (xyz.T.reshape(3, NCH, CH).swapaxes(0, 1).reshape(NCH, 3 * CH))
    return _tri_sc(planes, xyzp).reshape(M, CHAN)


# speedup vs baseline: 1.1292x; 1.1292x over previous
import functools

import jax
import jax.numpy as jnp
from jax import lax
from jax.experimental import pallas as pl
from jax.experimental.pallas import tpu as pltpu
from jax.experimental.pallas import tpu_sc as plsc

RESO = 512
CHAN = 32
M = 1000000

NC = 2
NS = 16
L = 16
NW = NC * NS

CH = 80
NCH = M // CH
ITERS = -(-NCH // NW)
NSTEP = ITERS + 1
NSRC = 12

_mesh = plsc.VectorSubcoreMesh(core_axis_name="c", subcore_axis_name="s")


@functools.partial(
    pl.kernel,
    mesh=_mesh,
    out_type=jax.ShapeDtypeStruct((M * CHAN,), jnp.float32),
    scratch_types=[
        pltpu.VMEM((2, 3 * CH), jnp.float32),
        pltpu.VMEM((2, NSRC, CH), jnp.int32),
        pltpu.VMEM((2, CH * 17), jnp.float32),
        pltpu.VMEM((2, NSRC, CH, CHAN), jnp.float32),
        pltpu.VMEM((2, CH * CHAN), jnp.float32),
        pltpu.SemaphoreType.DMA,
        pltpu.SemaphoreType.DMA,
        pltpu.SemaphoreType.DMA,
        pltpu.SemaphoreType.DMA,
        pltpu.SemaphoreType.DMA,
        pltpu.SemaphoreType.DMA,
    ],
    compiler_params=pltpu.CompilerParams(needs_layout_passes=False,
                                         use_tc_tiling_on_sc=False),
)
def _tri_sc(planes_hbm, xyzp_hbm, out_hbm, xyz_v, idx_v, w_v,
            rows_v, out_v, sem0, sem1, semx0, semx1, semo0, semo1):
    wid = lax.axis_index("s") * NC + lax.axis_index("c")
    sems = (sem0, sem1)
    semx = (semx0, semx1)
    semo = (semo0, semo1)
    stride16 = lax.iota(jnp.int32, L) * CHAN
    iota16 = lax.iota(jnp.int32, L)

    def bf16_round(v):
        u = lax.bitcast_convert_type(v, jnp.uint32)
        u = ((u + jnp.uint32(0x7FFF) + ((u >> 16) & jnp.uint32(1)))
             & jnp.uint32(0xFFFF0000))
        return lax.bitcast_convert_type(u, jnp.float32)

    def prefetch_xyz(it, b):
        ch = it * NW + wid

        @pl.when(ch < NCH)
        def _():
            pltpu.async_copy(xyzp_hbm.at[ch], xyz_v.at[b], semx[b])

    def fire(it, b):
        ch = it * NW + wid
        prefetch_xyz(it + 1, 1 - b)

        @pl.when(ch < NCH)
        def _():
            pltpu.make_async_copy(xyzp_hbm.at[ch], xyz_v.at[b],
                                  semx[b]).wait()

            for g in range(CH // L):
                sl = pl.ds(g * L, L)
                x = bf16_round(xyz_v[b, pl.ds(0 * CH + g * L, L)])
                y = bf16_round(xyz_v[b, pl.ds(1 * CH + g * L, L)])
                z = bf16_round(xyz_v[b, pl.ds(2 * CH + g * L, L)])
                for p, (gx, gy) in enumerate(((y, x), (z, x), (y, z))):
                    colf = (gx + 1.0) * (0.5 * (RESO - 1))
                    rowf = (gy + 1.0) * (0.5 * (RESO - 1))
                    c0 = jnp.clip(colf.astype(jnp.int32), 0, RESO - 2)
                    r0 = jnp.clip(rowf.astype(jnp.int32), 0, RESO - 2)
                    fc = colf - c0.astype(jnp.float32)
                    fr = rowf - r0.astype(jnp.float32)
                    base_i = r0 * RESO + c0 + (p * RESO * RESO)
                    idx_v[b, 4 * p + 0, sl] = base_i
                    idx_v[b, 4 * p + 1, sl] = base_i + 1
                    idx_v[b, 4 * p + 2, sl] = base_i + RESO
                    idx_v[b, 4 * p + 3, sl] = base_i + RESO + 1
                    wsc = iota16 * 17 + (g * L * 17 + 4 * p)
                    plsc.store_scatter(w_v.at[b], [wsc],
                                       (1.0 - fc) * (1.0 - fr))
                    plsc.store_scatter(w_v.at[b], [wsc + 1], fc * (1.0 - fr))
                    plsc.store_scatter(w_v.at[b], [wsc + 2], (1.0 - fc) * fr)
                    plsc.store_scatter(w_v.at[b], [wsc + 3], fc * fr)

            for s in range(NSRC):
                pltpu.async_copy(planes_hbm.at[idx_v.at[b, s]],
                                 rows_v.at[b, s], sems[b])

    def drain_acc(it, b):
        ch = it * NW + wid

        @pl.when(jnp.logical_and(it >= 0, ch < NCH))
        def _():
            base = ch * CH
            for s in range(NSRC):
                pltpu.make_async_copy(planes_hbm.at[idx_v.at[b, s]],
                                      rows_v.at[b, s], sems[b]).wait()

            @pl.when(it >= 2)
            def _wait_out():
                pltpu.make_async_copy(
                    out_v.at[b],
                    out_hbm.at[pl.ds(base * CHAN, CH * CHAN)],
                    semo[b]).wait()

            def pt_body(pt, carry2):
                o = pt * CHAN
                wrow = w_v[b, pl.ds(pt * 17, L)]
                w = wrow[0]
                acc0 = w * rows_v[b, 0, pt, pl.ds(0, L)]
                acc1 = w * rows_v[b, 0, pt, pl.ds(L, L)]
                for s in range(1, NSRC):
                    w = wrow[s]
                    acc0 = acc0 + w * rows_v[b, s, pt, pl.ds(0, L)]
                    acc1 = acc1 + w * rows_v[b, s, pt, pl.ds(L, L)]
                out_v[b, pl.ds(o, L)] = acc0
                out_v[b, pl.ds(o + L, L)] = acc1
                return carry2

            lax.fori_loop(0, CH, pt_body, 0, unroll=4)

            pltpu.async_copy(out_v.at[b],
                             out_hbm.at[pl.ds(base * CHAN, CH * CHAN)],
                             semo[b])

    def pair_body(it2, carry):
        for parity in range(2):
            step = it2 * 2 + parity
            fire(step, parity)
            drain_acc(step - 1, 1 - parity)
        return carry

    prefetch_xyz(0, 0)
    lax.fori_loop(0, NSTEP // 2, pair_body, 0)
    for b in range(2):
        pltpu.make_async_copy(out_v.at[b], out_hbm.at[pl.ds(0, CH * CHAN)],
                              semo[b]).wait()


def kernel(xyz, triplane):
    planes = jnp.transpose(triplane, (0, 2, 3, 1)).reshape(3 * RESO * RESO,
                                                           CHAN)
    xyzp = (xyz.T.reshape(3, NCH, CH).swapaxes(0, 1).reshape(NCH, 3 * CH))
    return _tri_sc(planes, xyzp).reshape(M, CHAN)


# speedup vs baseline: 1.2206x; 1.0809x over previous
import functools

import jax
import jax.numpy as jnp
from jax import lax
from jax.experimental import pallas as pl
from jax.experimental.pallas import tpu as pltpu
from jax.experimental.pallas import tpu_sc as plsc

RESO = 512
CHAN = 32
M = 1000000

NC = 2
NS = 16
L = 16
NW = NC * NS

CH = 80
NCH = M // CH
ITERS = -(-NCH // NW)
NSTEP = ITERS + 1
NSRC = 12

_mesh = plsc.VectorSubcoreMesh(core_axis_name="c", subcore_axis_name="s")


@functools.partial(
    pl.kernel,
    mesh=_mesh,
    out_type=jax.ShapeDtypeStruct((M * CHAN,), jnp.float32),
    scratch_types=[
        pltpu.VMEM((2, 3 * CH), jnp.float32),
        pltpu.VMEM((2, NSRC, CH), jnp.int32),
        pltpu.VMEM((2, CH * 17), jnp.float32),
        pltpu.VMEM((2, NSRC, CH, CHAN), jnp.float32),
        pltpu.VMEM((2, CH * CHAN), jnp.float32),
        pltpu.SemaphoreType.DMA,
        pltpu.SemaphoreType.DMA,
        pltpu.SemaphoreType.DMA,
        pltpu.SemaphoreType.DMA,
        pltpu.SemaphoreType.DMA,
        pltpu.SemaphoreType.DMA,
    ],
    compiler_params=pltpu.CompilerParams(needs_layout_passes=False,
                                         use_tc_tiling_on_sc=False),
)
def _tri_sc(planes_hbm, x_hbm, y_hbm, z_hbm, out_hbm, xyz_v, idx_v, w_v,
            rows_v, out_v, sem0, sem1, semx0, semx1, semo0, semo1):
    wid = lax.axis_index("s") * NC + lax.axis_index("c")
    sems = (sem0, sem1)
    semx = (semx0, semx1)
    semo = (semo0, semo1)
    stride16 = lax.iota(jnp.int32, L) * CHAN
    iota16 = lax.iota(jnp.int32, L)

    def bf16_round(v):
        u = lax.bitcast_convert_type(v, jnp.uint32)
        u = ((u + jnp.uint32(0x7FFF) + ((u >> 16) & jnp.uint32(1)))
             & jnp.uint32(0xFFFF0000))
        return lax.bitcast_convert_type(u, jnp.float32)

    def prefetch_xyz(it, b):
        ch = it * NW + wid

        @pl.when(ch < NCH)
        def _():
            base = ch * CH
            for i, coord in enumerate((x_hbm, y_hbm, z_hbm)):
                pltpu.async_copy(coord.at[pl.ds(base, CH)],
                                 xyz_v.at[b, pl.ds(i * CH, CH)], semx[b])

    def fire(it, b):
        ch = it * NW + wid
        prefetch_xyz(it + 1, 1 - b)

        @pl.when(ch < NCH)
        def _():
            base0 = ch * CH
            for i, coord in enumerate((x_hbm, y_hbm, z_hbm)):
                pltpu.make_async_copy(coord.at[pl.ds(base0, CH)],
                                      xyz_v.at[b, pl.ds(i * CH, CH)],
                                      semx[b]).wait()

            for g in range(CH // L):
                sl = pl.ds(g * L, L)
                x = bf16_round(xyz_v[b, pl.ds(0 * CH + g * L, L)])
                y = bf16_round(xyz_v[b, pl.ds(1 * CH + g * L, L)])
                z = bf16_round(xyz_v[b, pl.ds(2 * CH + g * L, L)])
                for p, (gx, gy) in enumerate(((y, x), (z, x), (y, z))):
                    colf = (gx + 1.0) * (0.5 * (RESO - 1))
                    rowf = (gy + 1.0) * (0.5 * (RESO - 1))
                    c0 = jnp.clip(colf.astype(jnp.int32), 0, RESO - 2)
                    r0 = jnp.clip(rowf.astype(jnp.int32), 0, RESO - 2)
                    fc = colf - c0.astype(jnp.float32)
                    fr = rowf - r0.astype(jnp.float32)
                    base_i = r0 * RESO + c0 + (p * RESO * RESO)
                    idx_v[b, 4 * p + 0, sl] = base_i
                    idx_v[b, 4 * p + 1, sl] = base_i + 1
                    idx_v[b, 4 * p + 2, sl] = base_i + RESO
                    idx_v[b, 4 * p + 3, sl] = base_i + RESO + 1
                    wsc = iota16 * 17 + (g * L * 17 + 4 * p)
                    plsc.store_scatter(w_v.at[b], [wsc],
                                       (1.0 - fc) * (1.0 - fr))
                    plsc.store_scatter(w_v.at[b], [wsc + 1], fc * (1.0 - fr))
                    plsc.store_scatter(w_v.at[b], [wsc + 2], (1.0 - fc) * fr)
                    plsc.store_scatter(w_v.at[b], [wsc + 3], fc * fr)

            for s in range(NSRC):
                pltpu.async_copy(planes_hbm.at[idx_v.at[b, s]],
                                 rows_v.at[b, s], sems[b])

    def drain_acc(it, b):
        ch = it * NW + wid

        @pl.when(jnp.logical_and(it >= 0, ch < NCH))
        def _():
            base = ch * CH
            for s in range(NSRC):
                pltpu.make_async_copy(planes_hbm.at[idx_v.at[b, s]],
                                      rows_v.at[b, s], sems[b]).wait()

            @pl.when(it >= 2)
            def _wait_out():
                pltpu.make_async_copy(
                    out_v.at[b],
                    out_hbm.at[pl.ds(base * CHAN, CH * CHAN)],
                    semo[b]).wait()

            def pt_body(pt, carry2):
                o = pt * CHAN
                wrow = w_v[b, pl.ds(pt * 17, L)]
                w = wrow[0]
                acc0 = w * rows_v[b, 0, pt, pl.ds(0, L)]
                acc1 = w * rows_v[b, 0, pt, pl.ds(L, L)]
                for s in range(1, NSRC):
                    w = wrow[s]
                    acc0 = acc0 + w * rows_v[b, s, pt, pl.ds(0, L)]
                    acc1 = acc1 + w * rows_v[b, s, pt, pl.ds(L, L)]
                out_v[b, pl.ds(o, L)] = acc0
                out_v[b, pl.ds(o + L, L)] = acc1
                return carry2

            lax.fori_loop(0, CH, pt_body, 0, unroll=8)

            pltpu.async_copy(out_v.at[b],
                             out_hbm.at[pl.ds(base * CHAN, CH * CHAN)],
                             semo[b])

    def pair_body(it2, carry):
        for parity in range(2):
            step = it2 * 2 + parity
            fire(step, parity)
            drain_acc(step - 1, 1 - parity)
        return carry

    prefetch_xyz(0, 0)
    lax.fori_loop(0, NSTEP // 2, pair_body, 0)
    for b in range(2):
        pltpu.make_async_copy(out_v.at[b], out_hbm.at[pl.ds(0, CH * CHAN)],
                              semo[b]).wait()


def kernel(xyz, triplane):
    planes = jnp.transpose(triplane, (0, 2, 3, 1)).reshape(3 * RESO * RESO,
                                                           CHAN)
    return _tri_sc(planes, xyz[:, 0], xyz[:, 1],
                   xyz[:, 2]).reshape(M, CHAN)


# speedup vs baseline: 1.2211x; 1.0003x over previous
import functools

import jax
import jax.numpy as jnp
from jax import lax
from jax.experimental import pallas as pl
from jax.experimental.pallas import tpu as pltpu
from jax.experimental.pallas import tpu_sc as plsc

RESO = 512
CHAN = 32
M = 1000000

NC = 2
NS = 16
L = 16
NW = NC * NS

CH = 80
NCH = M // CH
ITERS = -(-NCH // NW)
NSTEP = ITERS + 1
NSRC = 12

_mesh = plsc.VectorSubcoreMesh(core_axis_name="c", subcore_axis_name="s")


@functools.partial(
    pl.kernel,
    mesh=_mesh,
    out_type=jax.ShapeDtypeStruct((M * CHAN,), jnp.float32),
    scratch_types=[
        pltpu.VMEM((2, 3 * CH), jnp.float32),
        pltpu.VMEM((2, NSRC, CH), jnp.int32),
        pltpu.VMEM((2, CH * 17), jnp.float32),
        pltpu.VMEM((2, NSRC, CH, CHAN), jnp.float32),
        pltpu.VMEM((2, CH * CHAN), jnp.float32),
        pltpu.SemaphoreType.DMA,
        pltpu.SemaphoreType.DMA,
        pltpu.SemaphoreType.DMA,
        pltpu.SemaphoreType.DMA,
        pltpu.SemaphoreType.DMA,
        pltpu.SemaphoreType.DMA,
    ],
    compiler_params=pltpu.CompilerParams(needs_layout_passes=False,
                                         use_tc_tiling_on_sc=False),
)
def _tri_sc(planes_hbm, x_hbm, y_hbm, z_hbm, out_hbm, xyz_v, idx_v, w_v,
            rows_v, out_v, sem0, sem1, semx0, semx1, semo0, semo1):
    wid = lax.axis_index("s") * NC + lax.axis_index("c")
    sems = (sem0, sem1)
    semx = (semx0, semx1)
    semo = (semo0, semo1)
    iota16 = lax.iota(jnp.int32, L)

    def bf16_round(v):
        u = lax.bitcast_convert_type(v, jnp.uint32)
        u = ((u + jnp.uint32(0x7FFF) + ((u >> 16) & jnp.uint32(1)))
             & jnp.uint32(0xFFFF0000))
        return lax.bitcast_convert_type(u, jnp.float32)

    def prefetch_xyz(it, b):
        ch = it * NW + wid

        @pl.when(ch < NCH)
        def _():
            base = ch * CH
            for i, coord in enumerate((x_hbm, y_hbm, z_hbm)):
                pltpu.async_copy(coord.at[pl.ds(base, CH)],
                                 xyz_v.at[b, pl.ds(i * CH, CH)], semx[b])

    def fire(it, b):
        ch = it * NW + wid
        prefetch_xyz(it + 1, 1 - b)

        @pl.when(ch < NCH)
        def _():
            base0 = ch * CH
            for i, coord in enumerate((x_hbm, y_hbm, z_hbm)):
                pltpu.make_async_copy(coord.at[pl.ds(base0, CH)],
                                      xyz_v.at[b, pl.ds(i * CH, CH)],
                                      semx[b]).wait()

            for g in range(CH // L):
                sl = pl.ds(g * L, L)
                x = bf16_round(xyz_v[b, pl.ds(0 * CH + g * L, L)])
                y = bf16_round(xyz_v[b, pl.ds(1 * CH + g * L, L)])
                z = bf16_round(xyz_v[b, pl.ds(2 * CH + g * L, L)])
                for p, (gx, gy) in enumerate(((y, x), (z, x), (y, z))):
                    colf = (gx + 1.0) * (0.5 * (RESO - 1))
                    rowf = (gy + 1.0) * (0.5 * (RESO - 1))
                    c0 = jnp.clip(colf.astype(jnp.int32), 0, RESO - 2)
                    r0 = jnp.clip(rowf.astype(jnp.int32), 0, RESO - 2)
                    fc = colf - c0.astype(jnp.float32)
                    fr = rowf - r0.astype(jnp.float32)
                    base_i = r0 * RESO + c0 + (p * RESO * RESO)
                    idx_v[b, 4 * p + 0, sl] = base_i
                    idx_v[b, 4 * p + 1, sl] = base_i + 1
                    idx_v[b, 4 * p + 2, sl] = base_i + RESO
                    idx_v[b, 4 * p + 3, sl] = base_i + RESO + 1
                    wsc = iota16 * 17 + (g * L * 17 + 4 * p)
                    plsc.store_scatter(w_v.at[b], [wsc],
                                       (1.0 - fc) * (1.0 - fr))
                    plsc.store_scatter(w_v.at[b], [wsc + 1], fc * (1.0 - fr))
                    plsc.store_scatter(w_v.at[b], [wsc + 2], (1.0 - fc) * fr)
                    plsc.store_scatter(w_v.at[b], [wsc + 3], fc * fr)

            for s in range(NSRC):
                pltpu.async_copy(planes_hbm.at[idx_v.at[b, s]],
                                 rows_v.at[b, s], sems[b])

    def drain_acc(it, b):
        ch = it * NW + wid

        @pl.when(jnp.logical_and(it >= 0, ch < NCH))
        def _():
            base = ch * CH
            for s in range(NSRC):
                pltpu.make_async_copy(planes_hbm.at[idx_v.at[b, s]],
                                      rows_v.at[b, s], sems[b]).wait()

            @pl.when(it >= 2)
            def _wait_out():
                pltpu.make_async_copy(
                    out_v.at[b],
                    out_hbm.at[pl.ds(base * CHAN, CH * CHAN)],
                    semo[b]).wait()

            def pt_body(pt, carry2):
                o = pt * CHAN
                wrow = w_v[b, pl.ds(pt * 17, L)]
                w = wrow[0]
                acc0 = w * rows_v[b, 0, pt, pl.ds(0, L)]
                acc1 = w * rows_v[b, 0, pt, pl.ds(L, L)]
                for s in range(1, NSRC):
                    w = wrow[s]
                    acc0 = acc0 + w * rows_v[b, s, pt, pl.ds(0, L)]
                    acc1 = acc1 + w * rows_v[b, s, pt, pl.ds(L, L)]
                out_v[b, pl.ds(o, L)] = acc0
                out_v[b, pl.ds(o + L, L)] = acc1
                return carry2

            lax.fori_loop(0, CH, pt_body, 0, unroll=8)

            pltpu.async_copy(out_v.at[b],
                             out_hbm.at[pl.ds(base * CHAN, CH * CHAN)],
                             semo[b])

    def pair_body(it2, carry):
        for parity in range(2):
            step = it2 * 2 + parity
            fire(step, parity)
            drain_acc(step - 1, 1 - parity)
        return carry

    prefetch_xyz(0, 0)
    lax.fori_loop(0, NSTEP // 2, pair_body, 0)
    for b in range(2):
        pltpu.make_async_copy(out_v.at[b], out_hbm.at[pl.ds(0, CH * CHAN)],
                              semo[b]).wait()


def kernel(xyz, triplane):
    planes = jnp.transpose(triplane, (0, 2, 3, 1)).reshape(3 * RESO * RESO,
                                                           CHAN)
    return _tri_sc(planes, xyz[:, 0], xyz[:, 1],
                   xyz[:, 2]).reshape(M, CHAN)
